# Initial kernel scaffold; baseline (speedup 1.0000x reference)
import jax
import jax.numpy as jnp
from jax.experimental import pallas as pl


def kernel(input_ids, elapsed_time, product_action, hashed_url, price_bucket, number_of_category_hash, category_hash_first_level, category_hash_second_level, category_hash_third_level, id_table, time_table, action_table, url_table, price_table, numcat_table, cat1_table, cat2_table, cat3_table, W, b, gamma, beta):
    raise NotImplementedError("write your pallas kernel here")


# speedup vs baseline: 2.0440x; 2.0440x over previous
import functools

import jax
import jax.numpy as jnp
from jax import lax
from jax.experimental import pallas as pl
from jax.experimental.pallas import tpu as pltpu
from jax.experimental.pallas import tpu_sc as plsc

B, L, EMB, HID = 4096, 50, 64, 256
N = B * L
NT = 9
NC, NS = 2, 16
NW = NC * NS
CHUNK = 128
TOK_PER_W = N // NW
CHUNKS_PER_W = TOK_PER_W // CHUNK


def _sc_gather_body(idx_hbm, t0, t1, t2, t3, t4, t5, t6, t7, t8,
                    x_hbm, idx_v, rows_v, sem):
    tables = (t0, t1, t2, t3, t4, t5, t6, t7, t8)
    wid = lax.axis_index("s") * NC + lax.axis_index("c")
    base_tok = wid * TOK_PER_W
    for t in range(NT):
        pltpu.sync_copy(idx_hbm.at[t, wid], idx_v)
        table = tables[t]

        def body(j, carry, table=table, t=t):
            pltpu.async_copy(table.at[idx_v.at[j]], rows_v, sem).wait()
            pltpu.sync_copy(
                rows_v, x_hbm.at[t, pl.ds(base_tok + j * CHUNK, CHUNK)])
            return carry

        lax.fori_loop(0, CHUNKS_PER_W, body, 0)


def _sc_gather(idx, tables):
    mesh = plsc.VectorSubcoreMesh(core_axis_name="c", subcore_axis_name="s")
    f = functools.partial(
        pl.kernel,
        out_type=jax.ShapeDtypeStruct((NT, N, 2 * EMB), jnp.float32),
        mesh=mesh,
        scratch_types=[
            pltpu.VMEM((CHUNKS_PER_W, CHUNK), jnp.int32),
            pltpu.VMEM((CHUNK, 2 * EMB), jnp.float32),
            pltpu.SemaphoreType.DMA,
        ],
    )(_sc_gather_body)
    return f(idx, *tables)


def _tc_body(x_ref, w_ref, b_ref, g_ref, be_ref, o_ref):
    xb = x_ref[...]
    xcat = jnp.concatenate([xb[t, :, :EMB] for t in range(NT)], axis=1)
    h = jnp.dot(xcat, w_ref[...], preferred_element_type=jnp.float32)
    h = h + b_ref[...]
    mu = jnp.mean(h, axis=1, keepdims=True)
    d = h - mu
    var = jnp.mean(d * d, axis=1, keepdims=True)
    o_ref[...] = d * lax.rsqrt(var + 1e-12) * g_ref[...] + be_ref[...]


def _tc_proj_ln(x, W, b, gamma, beta, block=512):
    grid = (N // block,)
    return pl.pallas_call(
        _tc_body,
        grid=grid,
        in_specs=[
            pl.BlockSpec((NT, block, 2 * EMB), lambda i: (0, i, 0)),
            pl.BlockSpec((NT * EMB, HID), lambda i: (0, 0)),
            pl.BlockSpec((1, HID), lambda i: (0, 0)),
            pl.BlockSpec((1, HID), lambda i: (0, 0)),
            pl.BlockSpec((1, HID), lambda i: (0, 0)),
        ],
        out_specs=pl.BlockSpec((block, HID), lambda i: (i, 0)),
        out_shape=jax.ShapeDtypeStruct((N, HID), jnp.float32),
    )(x, W, b.reshape(1, HID), gamma.reshape(1, HID), beta.reshape(1, HID))


def kernel(input_ids, elapsed_time, product_action, hashed_url, price_bucket,
           number_of_category_hash, category_hash_first_level,
           category_hash_second_level, category_hash_third_level,
           id_table, time_table, action_table, url_table, price_table,
           numcat_table, cat1_table, cat2_table, cat3_table,
           W, b, gamma, beta):
    elapsed_cat = jnp.clip(elapsed_time.astype(jnp.int32) + 1, 0, 10000)
    idx = jnp.stack([
        input_ids.reshape(-1), elapsed_cat.reshape(-1),
        product_action.reshape(-1), hashed_url.reshape(-1),
        price_bucket.reshape(-1), number_of_category_hash.reshape(-1),
        category_hash_first_level.reshape(-1),
        category_hash_second_level.reshape(-1),
        category_hash_third_level.reshape(-1),
    ]).astype(jnp.int32).reshape(NT, NW, CHUNKS_PER_W, CHUNK)
    tables = (id_table, time_table, action_table, url_table, price_table,
              numcat_table, cat1_table, cat2_table, cat3_table)
    tables = tuple(jnp.pad(t, ((0, 0), (0, EMB))) for t in tables)
    x = _sc_gather(idx, tables)
    out = _tc_proj_ln(x, W, b, gamma, beta)
    return out.reshape(B, L, HID)


# speedup vs baseline: 2.1479x; 1.0508x over previous
import functools

import jax
import jax.numpy as jnp
from jax import lax
from jax.experimental import pallas as pl
from jax.experimental.pallas import tpu as pltpu
from jax.experimental.pallas import tpu_sc as plsc

B, L, EMB, HID = 4096, 50, 64, 256
N = B * L
NT = 9
NC, NS = 2, 16
NW = NC * NS
CHUNK = 128
TOK_PER_W = N // NW
CHUNKS_PER_W = TOK_PER_W // CHUNK


NBUF = 5
ROUNDS = CHUNKS_PER_W // NBUF


def _sc_gather_body(idx_hbm, t0, t1, t2, t3, t4, t5, t6, t7, t8,
                    x_hbm, idx_v, rows, gsems, wsems):
    tables = (t0, t1, t2, t3, t4, t5, t6, t7, t8)
    wid = lax.axis_index("s") * NC + lax.axis_index("c")
    base_tok = wid * TOK_PER_W
    for t in range(NT):
        pltpu.sync_copy(idx_hbm.at[t, wid], idx_v)
        table = tables[t]

        def gather(j, b, table=table):
            pltpu.async_copy(table.at[idx_v.at[j]], rows.at[b], gsems.at[b])

        def gather_wait(b, table=table):
            pltpu.make_async_copy(table.at[idx_v.at[0]], rows.at[b],
                                  gsems.at[b]).wait()

        def write(j, b, t=t):
            pltpu.async_copy(
                rows.at[b], x_hbm.at[t, pl.ds(base_tok + j * CHUNK, CHUNK)],
                wsems.at[b])

        def write_wait(b, t=t):
            pltpu.make_async_copy(
                rows.at[b], x_hbm.at[t, pl.ds(base_tok, CHUNK)],
                wsems.at[b]).wait()

        for b in range(NBUF):
            gather(b, b)

        def body(j2, carry):
            for b in range(NBUF):
                gather_wait(b)
                write(j2 * NBUF + b, b)
            for b in range(NBUF):
                @pl.when(j2 < ROUNDS - 1)
                def _(b=b):
                    write_wait(b)
                    gather((j2 + 1) * NBUF + b, b)
            return carry

        lax.fori_loop(0, ROUNDS, body, 0)
        for b in range(NBUF):
            write_wait(b)


def _sc_gather(idx, tables):
    mesh = plsc.VectorSubcoreMesh(core_axis_name="c", subcore_axis_name="s")
    f = functools.partial(
        pl.kernel,
        out_type=jax.ShapeDtypeStruct((NT, N, 2 * EMB), jnp.float32),
        mesh=mesh,
        scratch_types=[
            pltpu.VMEM((CHUNKS_PER_W, CHUNK), jnp.int32),
            pltpu.VMEM((NBUF, CHUNK, 2 * EMB), jnp.float32),
            pltpu.SemaphoreType.DMA((NBUF,)),
            pltpu.SemaphoreType.DMA((NBUF,)),
        ],
    )(_sc_gather_body)
    return f(idx, *tables)


def _tc_body(x_ref, w_ref, b_ref, g_ref, be_ref, o_ref):
    xb = x_ref[...]
    xcat = jnp.concatenate([xb[t, :, :EMB] for t in range(NT)], axis=1)
    h = jnp.dot(xcat, w_ref[...], preferred_element_type=jnp.float32)
    h = h + b_ref[...]
    mu = jnp.mean(h, axis=1, keepdims=True)
    d = h - mu
    var = jnp.mean(d * d, axis=1, keepdims=True)
    o_ref[...] = d * lax.rsqrt(var + 1e-12) * g_ref[...] + be_ref[...]


def _tc_proj_ln(x, W, b, gamma, beta, block=512):
    grid = (N // block,)
    return pl.pallas_call(
        _tc_body,
        grid=grid,
        in_specs=[
            pl.BlockSpec((NT, block, 2 * EMB), lambda i: (0, i, 0)),
            pl.BlockSpec((NT * EMB, HID), lambda i: (0, 0)),
            pl.BlockSpec((1, HID), lambda i: (0, 0)),
            pl.BlockSpec((1, HID), lambda i: (0, 0)),
            pl.BlockSpec((1, HID), lambda i: (0, 0)),
        ],
        out_specs=pl.BlockSpec((block, HID), lambda i: (i, 0)),
        out_shape=jax.ShapeDtypeStruct((N, HID), jnp.float32),
    )(x, W, b.reshape(1, HID), gamma.reshape(1, HID), beta.reshape(1, HID))


def kernel(input_ids, elapsed_time, product_action, hashed_url, price_bucket,
           number_of_category_hash, category_hash_first_level,
           category_hash_second_level, category_hash_third_level,
           id_table, time_table, action_table, url_table, price_table,
           numcat_table, cat1_table, cat2_table, cat3_table,
           W, b, gamma, beta):
    elapsed_cat = jnp.clip(elapsed_time.astype(jnp.int32) + 1, 0, 10000)
    idx = jnp.stack([
        input_ids.reshape(-1), elapsed_cat.reshape(-1),
        product_action.reshape(-1), hashed_url.reshape(-1),
        price_bucket.reshape(-1), number_of_category_hash.reshape(-1),
        category_hash_first_level.reshape(-1),
        category_hash_second_level.reshape(-1),
        category_hash_third_level.reshape(-1),
    ]).astype(jnp.int32).reshape(NT, NW, CHUNKS_PER_W, CHUNK)
    tables = (id_table, time_table, action_table, url_table, price_table,
              numcat_table, cat1_table, cat2_table, cat3_table)
    tables = tuple(jnp.pad(t, ((0, 0), (0, EMB))) for t in tables)
    x = _sc_gather(idx, tables)
    out = _tc_proj_ln(x, W, b, gamma, beta)
    return out.reshape(B, L, HID)
